# baseline (speedup 1.0000x reference)
import jax
import jax.numpy as jnp
from jax.experimental import pallas as pl

_HIDDEN = 2048
_HALF = _HIDDEN // 2
_NE = 8


def _router_body(x_ref, w1_ref, b1_ref, w2_ref, b2_ref, logits_ref, idx_ref):
    h = jnp.dot(x_ref[:], w1_ref[:], preferred_element_type=jnp.float32)
    h = jnp.maximum(h + b1_ref[:], 0.0)
    logits = jnp.dot(h, w2_ref[:], preferred_element_type=jnp.float32)
    logits = logits + b2_ref[:]
    logits_ref[:] = logits
    m = jnp.max(logits, axis=1, keepdims=True)
    lane = jax.lax.broadcasted_iota(jnp.int32, logits.shape, 1)
    idx_ref[:] = jnp.min(jnp.where(logits == m, lane, _NE), axis=1, keepdims=True)


def kernel(hidden_states, W1, b1, W2, b2):
    B, S, H = hidden_states.shape
    M = B * S
    x = hidden_states.reshape(M, H)
    m_blk = 512
    grid = (M // m_blk,)

    logits, idx = pl.pallas_call(
        _router_body,
        grid=grid,
        in_specs=[
            pl.BlockSpec((m_blk, H), lambda i: (i, 0)),
            pl.BlockSpec((H, _HALF), lambda i: (0, 0)),
            pl.BlockSpec((1, _HALF), lambda i: (0, 0)),
            pl.BlockSpec((_HALF, _NE), lambda i: (0, 0)),
            pl.BlockSpec((1, _NE), lambda i: (0, 0)),
        ],
        out_specs=[
            pl.BlockSpec((m_blk, _NE), lambda i: (i, 0)),
            pl.BlockSpec((m_blk, 1), lambda i: (i, 0)),
        ],
        out_shape=[
            jax.ShapeDtypeStruct((M, _NE), jnp.float32),
            jax.ShapeDtypeStruct((M, 1), jnp.int32),
        ],
    )(x, W1, b1.reshape(1, _HALF), W2, b2.reshape(1, _NE))

    return idx.reshape(B, S), logits.reshape(B, S, _NE)


# speedup vs baseline: 1.0746x; 1.0746x over previous
import jax
import jax.numpy as jnp
from jax.experimental import pallas as pl

_HIDDEN = 2048
_HALF = _HIDDEN // 2
_NE = 8


def _router_body(x_ref, w1_ref, b1_ref, w2_ref, b2_ref, logits_ref, idx_ref):
    h = jnp.dot(x_ref[:], w1_ref[:], preferred_element_type=jnp.float32)
    h = jnp.maximum(h + b1_ref[:], 0.0)
    logits = jnp.dot(h, w2_ref[:], preferred_element_type=jnp.float32)
    logits = logits + b2_ref[:]
    logits_ref[:] = logits
    m = jnp.max(logits, axis=1, keepdims=True)
    lane = jax.lax.broadcasted_iota(jnp.int32, logits.shape, 1)
    idx_ref[:] = jnp.min(jnp.where(logits == m, lane, _NE), axis=1, keepdims=True)


def kernel(hidden_states, W1, b1, W2, b2):
    B, S, H = hidden_states.shape
    M = B * S
    x = hidden_states.reshape(M, H)
    m_blk = 1024
    grid = (M // m_blk,)

    logits, idx = pl.pallas_call(
        _router_body,
        grid=grid,
        in_specs=[
            pl.BlockSpec((m_blk, H), lambda i: (i, 0)),
            pl.BlockSpec((H, _HALF), lambda i: (0, 0)),
            pl.BlockSpec((1, _HALF), lambda i: (0, 0)),
            pl.BlockSpec((_HALF, _NE), lambda i: (0, 0)),
            pl.BlockSpec((1, _NE), lambda i: (0, 0)),
        ],
        out_specs=[
            pl.BlockSpec((m_blk, _NE), lambda i: (i, 0)),
            pl.BlockSpec((m_blk, 1), lambda i: (i, 0)),
        ],
        out_shape=[
            jax.ShapeDtypeStruct((M, _NE), jnp.float32),
            jax.ShapeDtypeStruct((M, 1), jnp.int32),
        ],
    )(x, W1, b1.reshape(1, _HALF), W2, b2.reshape(1, _NE))

    return idx.reshape(B, S), logits.reshape(B, S, _NE)


# speedup vs baseline: 1.1405x; 1.0613x over previous
import jax
import jax.numpy as jnp
from jax.experimental import pallas as pl

_HIDDEN = 2048
_HALF = _HIDDEN // 2
_NE = 8


def _router_body(x_ref, w1_ref, b1_ref, w2_ref, b2_ref, logits_ref, idx_ref):
    h = jnp.dot(x_ref[:], w1_ref[:], preferred_element_type=jnp.float32)
    h = jnp.maximum(h + b1_ref[:], 0.0)
    logits = jnp.dot(h, w2_ref[:], preferred_element_type=jnp.float32)
    logits = logits + b2_ref[:]
    logits_ref[:] = logits
    lt = logits.T
    m = jnp.max(lt, axis=0, keepdims=True)
    expert = jax.lax.broadcasted_iota(jnp.int32, lt.shape, 0)
    idx_ref[:] = jnp.min(jnp.where(lt == m, expert, _NE), axis=0)


def kernel(hidden_states, W1, b1, W2, b2):
    B, S, H = hidden_states.shape
    M = B * S
    x = hidden_states.reshape(M, H)
    m_blk = 1024
    grid = (M // m_blk,)

    logits, idx = pl.pallas_call(
        _router_body,
        grid=grid,
        in_specs=[
            pl.BlockSpec((m_blk, H), lambda i: (i, 0)),
            pl.BlockSpec((H, _HALF), lambda i: (0, 0)),
            pl.BlockSpec((1, _HALF), lambda i: (0, 0)),
            pl.BlockSpec((_HALF, _NE), lambda i: (0, 0)),
            pl.BlockSpec((1, _NE), lambda i: (0, 0)),
        ],
        out_specs=[
            pl.BlockSpec((m_blk, _NE), lambda i: (i, 0)),
            pl.BlockSpec((m_blk,), lambda i: (i,)),
        ],
        out_shape=[
            jax.ShapeDtypeStruct((M, _NE), jnp.float32),
            jax.ShapeDtypeStruct((M,), jnp.int32),
        ],
    )(x, W1, b1.reshape(1, _HALF), W2, b2.reshape(1, _NE))

    return idx.reshape(B, S), logits.reshape(B, S, _NE)
